# baseline (speedup 1.0000x reference)
import functools

import jax
import jax.numpy as jnp
from jax import lax
from jax.experimental import pallas as pl
from jax.experimental.pallas import tpu as pltpu
from jax.experimental.pallas import tpu_sc as plsc

DIM = 64
NUM_CORES = 2
NUM_SUBCORES = 16
NW = NUM_CORES * NUM_SUBCORES

CHUNK = 128
GATHERS = 8
GROUP = CHUNK * GATHERS


def _make_kernel(n_tokens: int):
    b_per_w = n_tokens // NW
    n_groups = b_per_w // GROUP

    mesh = plsc.VectorSubcoreMesh(core_axis_name="c", subcore_axis_name="s")

    @functools.partial(
        pl.kernel,
        mesh=mesh,
        out_type=jax.ShapeDtypeStruct((n_tokens, DIM), jnp.float32),
        scratch_types=[
            pltpu.VMEM((b_per_w,), jnp.int32),
            pltpu.VMEM((GROUP, DIM), jnp.float32),
            pltpu.SemaphoreType.DMA,
        ],
        compiler_params=pltpu.CompilerParams(use_tc_tiling_on_sc=False),
    )
    def emb(idx_hbm, table_hbm, out_hbm, idx_v, rows_v, sem):
        wid = lax.axis_index("s") * NUM_CORES + lax.axis_index("c")
        base = wid * b_per_w
        pltpu.sync_copy(idx_hbm.at[pl.ds(base, b_per_w)], idx_v)

        def group_body(g, carry):
            off = g * GROUP
            copies = [
                pltpu.async_copy(
                    table_hbm.at[idx_v.at[pl.ds(off + j * CHUNK, CHUNK)]],
                    rows_v.at[pl.ds(j * CHUNK, CHUNK)],
                    sem,
                )
                for j in range(GATHERS)
            ]
            for c in copies:
                c.wait()
            pltpu.sync_copy(rows_v, out_hbm.at[pl.ds(base + off, GROUP)])
            return carry

        lax.fori_loop(0, n_groups, group_body, 0)

    return emb


def kernel(x, table):
    batch, seq = x.shape
    n_tokens = batch * seq
    x_flat = x.reshape(n_tokens).astype(jnp.int32)
    out = _make_kernel(n_tokens)(x_flat, table)
    return out.reshape(batch, seq, DIM)


# speedup vs baseline: 1.0066x; 1.0066x over previous
import functools

import jax
import jax.numpy as jnp
from jax import lax
from jax.experimental import pallas as pl
from jax.experimental.pallas import tpu as pltpu
from jax.experimental.pallas import tpu_sc as plsc

DIM = 64
NUM_CORES = 2
NUM_SUBCORES = 16
NW = NUM_CORES * NUM_SUBCORES

CHUNK = 128
GATHERS = 2
GROUP = CHUNK * GATHERS
NBUF = 4


def _make_kernel(n_tokens: int):
    b_per_w = n_tokens // NW
    n_groups = b_per_w // GROUP
    n_outer = n_groups // NBUF

    mesh = plsc.VectorSubcoreMesh(core_axis_name="c", subcore_axis_name="s")

    @functools.partial(
        pl.kernel,
        mesh=mesh,
        out_type=jax.ShapeDtypeStruct((n_tokens, DIM), jnp.float32),
        scratch_types=[
            pltpu.VMEM((b_per_w,), jnp.int32),
            [pltpu.VMEM((GROUP, DIM), jnp.float32) for _ in range(NBUF)],
            [pltpu.SemaphoreType.DMA for _ in range(NBUF)],
            [pltpu.SemaphoreType.DMA for _ in range(NBUF)],
        ],
        compiler_params=pltpu.CompilerParams(use_tc_tiling_on_sc=False),
    )
    def emb(idx_hbm, table_hbm, out_hbm, idx_v, bufs, gsems, wsems):
        wid = lax.axis_index("s") * NUM_CORES + lax.axis_index("c")
        base = wid * b_per_w
        pltpu.sync_copy(idx_hbm.at[pl.ds(base, b_per_w)], idx_v)

        def fire_gathers(g, b):
            off = g * GROUP
            for j in range(GATHERS):
                pltpu.async_copy(
                    table_hbm.at[idx_v.at[pl.ds(off + j * CHUNK, CHUNK)]],
                    bufs[b].at[pl.ds(j * CHUNK, CHUNK)],
                    gsems[b],
                )

        def drain_gathers(b):
            for j in range(GATHERS):
                pltpu.make_async_copy(
                    table_hbm.at[idx_v.at[pl.ds(0, CHUNK)]],
                    bufs[b].at[pl.ds(j * CHUNK, CHUNK)],
                    gsems[b],
                ).wait()

        def drain_write(b):
            pltpu.make_async_copy(
                bufs[b], out_hbm.at[pl.ds(0, GROUP)], wsems[b]
            ).wait()

        def outer(o, carry):
            for b in range(NBUF):
                g = o * NBUF + b

                @pl.when(o != 0)
                def _():
                    drain_write(b)

                fire_gathers(g, b)
            for b in range(NBUF):
                g = o * NBUF + b
                drain_gathers(b)
                pltpu.async_copy(
                    bufs[b], out_hbm.at[pl.ds(base + g * GROUP, GROUP)], wsems[b]
                )
            return carry

        lax.fori_loop(0, n_outer, outer, 0)
        for b in range(NBUF):
            drain_write(b)

    return emb


def kernel(x, table):
    batch, seq = x.shape
    n_tokens = batch * seq
    x_flat = x.reshape(n_tokens).astype(jnp.int32)
    out = _make_kernel(n_tokens)(x_flat, table)
    return out.reshape(batch, seq, DIM)


# speedup vs baseline: 1.1547x; 1.1471x over previous
import functools

import jax
import jax.numpy as jnp
from jax import lax
from jax.experimental import pallas as pl
from jax.experimental.pallas import tpu as pltpu
from jax.experimental.pallas import tpu_sc as plsc

VOCAB = 1000000
DIM = 64
NUM_CORES = 2
NUM_SUBCORES = 16
NW = NUM_CORES * NUM_SUBCORES
GRP = 128


def _make_gather(n_tokens: int):
    per_w = n_tokens // NW
    n_grp = per_w // GRP
    mesh = plsc.VectorSubcoreMesh(core_axis_name="c", subcore_axis_name="s")

    @functools.partial(
        pl.kernel,
        mesh=mesh,
        out_type=jax.ShapeDtypeStruct((n_tokens, DIM), jnp.float32),
        scratch_types=[
            pltpu.VMEM((per_w,), jnp.int32),
            [pltpu.VMEM((GRP,), jnp.int32) for _ in range(2)],
            [pltpu.VMEM((GRP,), jnp.int32) for _ in range(2)],
            [pltpu.VMEM((GRP, 2 * DIM), jnp.float32) for _ in range(2)],
            [pltpu.VMEM((GRP, DIM), jnp.float32) for _ in range(2)],
            [pltpu.SemaphoreType.DMA for _ in range(2)],
            [pltpu.SemaphoreType.DMA for _ in range(2)],
        ],
        compiler_params=pltpu.CompilerParams(
            use_tc_tiling_on_sc=True, needs_layout_passes=False
        ),
    )
    def kgat(xf, tab, out, ixbuf, qbuf, hbuf, stag, tbuf, gsem, wsem):
        wid = lax.axis_index("s") * NUM_CORES + lax.axis_index("c")
        base = wid * per_w
        iota = lax.iota(jnp.int32, 16)

        pltpu.sync_copy(xf.at[pl.ds(base, per_w)], ixbuf)

        def prep(g, b):
            for j in range(8):
                ids = ixbuf[pl.ds(g * GRP + j * 16, 16)]
                qbuf[b][pl.ds(j * 16, 16)] = lax.shift_right_logical(ids, 1)
                hbuf[b][pl.ds(j * 16, 16)] = lax.shift_left(
                    lax.bitwise_and(ids, 1), 6
                )

        def fire_gather(b):
            pltpu.async_copy(tab.at[qbuf[b]], stag[b], gsem[b])

        def wait_gather(b):
            pltpu.make_async_copy(tab.at[qbuf[b]], stag[b], gsem[b]).wait()

        def drain_write(b):
            pltpu.make_async_copy(
                tbuf[b], out.at[pl.ds(0, GRP)], wsem[b]
            ).wait()

        def extract(b):
            @plsc.parallel_loop(0, GRP, unroll=2)
            def _(k):
                ksplat = jnp.broadcast_to(k, (16,))
                hv = plsc.load_gather(hbuf[b], [ksplat])
                for c in range(DIM // 16):
                    idx = hv + (c * 16 + iota)
                    vals = plsc.load_gather(stag[b], [ksplat, idx])
                    tbuf[b][k, pl.ds(c * 16, 16)] = vals

        def fire_write(g, b):
            pltpu.async_copy(
                tbuf[b], out.at[pl.ds(base + g * GRP, GRP)], wsem[b]
            )

        prep(0, 0)
        fire_gather(0)

        def body2(i2, carry):
            for b in (0, 1):
                g = i2 * 2 + b

                @pl.when(g + 1 < n_grp)
                def _():
                    prep(g + 1, 1 - b)
                    fire_gather(1 - b)

                wait_gather(b)

                @pl.when(g >= 2)
                def _():
                    drain_write(b)

                extract(b)
                fire_write(g, b)

            return carry

        lax.fori_loop(0, n_grp // 2, body2, 0)
        drain_write(0)
        drain_write(1)

    return kgat


def kernel(x, table):
    batch, seq = x.shape
    n_tokens = batch * seq
    tab2 = table.reshape(VOCAB // 2, 2 * DIM)
    xf = x.reshape(n_tokens).astype(jnp.int32)
    out = _make_gather(n_tokens)(xf, tab2)
    return out.reshape(batch, seq, DIM)


# speedup vs baseline: 1.1567x; 1.0017x over previous
import functools

import jax
import jax.numpy as jnp
from jax import lax
from jax.experimental import pallas as pl
from jax.experimental.pallas import tpu as pltpu
from jax.experimental.pallas import tpu_sc as plsc

VOCAB = 1000000
DIM = 64
NUM_CORES = 2
NUM_SUBCORES = 16
NW = NUM_CORES * NUM_SUBCORES
GRP = 128


def _make_gather(n_tokens: int):
    per_w = n_tokens // NW
    n_grp = per_w // GRP
    mesh = plsc.VectorSubcoreMesh(core_axis_name="c", subcore_axis_name="s")

    @functools.partial(
        pl.kernel,
        mesh=mesh,
        out_type=jax.ShapeDtypeStruct((n_tokens, DIM), jnp.float32),
        scratch_types=[
            pltpu.VMEM((per_w,), jnp.int32),
            [pltpu.VMEM((GRP,), jnp.int32) for _ in range(2)],
            [pltpu.VMEM((GRP,), jnp.int32) for _ in range(2)],
            [pltpu.VMEM((GRP, 2 * DIM), jnp.float32) for _ in range(2)],
            [pltpu.VMEM((GRP, DIM), jnp.float32) for _ in range(2)],
            [pltpu.SemaphoreType.DMA for _ in range(2)],
            [pltpu.SemaphoreType.DMA for _ in range(2)],
        ],
        compiler_params=pltpu.CompilerParams(
            use_tc_tiling_on_sc=True, needs_layout_passes=False
        ),
    )
    def kgat(xf, tab, out, ixbuf, qbuf, hbuf, stag, tbuf, gsem, wsem):
        wid = lax.axis_index("s") * NUM_CORES + lax.axis_index("c")
        base = wid * per_w
        iota = lax.iota(jnp.int32, 16)

        pltpu.sync_copy(xf.at[pl.ds(base, per_w)], ixbuf)

        def prep(g, b):
            for j in range(8):
                ids = ixbuf[pl.ds(g * GRP + j * 16, 16)]
                qbuf[b][pl.ds(j * 16, 16)] = lax.shift_right_logical(ids, 1)
                hbuf[b][pl.ds(j * 16, 16)] = lax.shift_left(
                    lax.bitwise_and(ids, 1), 6
                )

        def fire_gather(b):
            pltpu.async_copy(tab.at[qbuf[b]], stag[b], gsem[b])

        def wait_gather(b):
            pltpu.make_async_copy(tab.at[qbuf[b]], stag[b], gsem[b]).wait()

        def drain_write(b):
            pltpu.make_async_copy(
                tbuf[b], out.at[pl.ds(0, GRP)], wsem[b]
            ).wait()

        def extract(b):
            @plsc.parallel_loop(0, GRP, unroll=4)
            def _(k):
                ksplat = jnp.broadcast_to(k, (16,))
                hv = plsc.load_gather(hbuf[b], [ksplat])
                for c in range(DIM // 16):
                    idx = hv + (c * 16 + iota)
                    vals = plsc.load_gather(stag[b], [ksplat, idx])
                    tbuf[b][k, pl.ds(c * 16, 16)] = vals

        def fire_write(g, b):
            pltpu.async_copy(
                tbuf[b], out.at[pl.ds(base + g * GRP, GRP)], wsem[b]
            )

        prep(0, 0)
        fire_gather(0)

        def body2(i2, carry):
            for b in (0, 1):
                g = i2 * 2 + b

                @pl.when(g + 1 < n_grp)
                def _():
                    prep(g + 1, 1 - b)
                    fire_gather(1 - b)

                wait_gather(b)

                @pl.when(g >= 2)
                def _():
                    drain_write(b)

                extract(b)
                fire_write(g, b)

            return carry

        lax.fori_loop(0, n_grp // 2, body2, 0)
        drain_write(0)
        drain_write(1)

    return kgat


def kernel(x, table):
    batch, seq = x.shape
    n_tokens = batch * seq
    tab2 = table.reshape(VOCAB // 2, 2 * DIM)
    xf = x.reshape(n_tokens).astype(jnp.int32)
    out = _make_gather(n_tokens)(xf, tab2)
    return out.reshape(batch, seq, DIM)
